# baseline (speedup 1.0000x reference)
import functools

import jax
import jax.numpy as jnp
from jax.experimental import pallas as pl
from jax.experimental.pallas import tpu as pltpu

_E = 8
_TOP_K = 2

_CB = 1024
_FB = 256


def _ffn_body(x_ref, s_ref, w1_ref, w2_ref, o_ref, acc_ref):
    f = pl.program_id(2)
    nf = pl.num_programs(2)

    x = x_ref[0] * s_ref[0, 0][:, None]
    xb = x.astype(jnp.bfloat16)
    h = jnp.dot(xb, w1_ref[0].astype(jnp.bfloat16),
                preferred_element_type=jnp.float32)
    h = h * jax.nn.sigmoid(h)
    part = jnp.dot(h.astype(jnp.bfloat16), w2_ref[0].astype(jnp.bfloat16),
                   preferred_element_type=jnp.float32)

    @pl.when(f == 0)
    def _():
        acc_ref[...] = part

    @pl.when(f != 0)
    def _():
        acc_ref[...] += part

    @pl.when(f == nf - 1)
    def _():
        o_ref[0] = acc_ref[...]


def _ffn(routed, scores, w1b, w2b, *, interpret=False):
    e, c, d = routed.shape
    dff = w1b.shape[2]
    cb = min(_CB, c)
    fb = min(_FB, dff)
    grid = (e, c // cb, dff // fb)
    return pl.pallas_call(
        _ffn_body,
        grid=grid,
        in_specs=[
            pl.BlockSpec((1, cb, d), lambda e, i, f: (e, i, 0)),
            pl.BlockSpec((1, 1, cb), lambda e, i, f: (e, 0, i)),
            pl.BlockSpec((1, d, fb), lambda e, i, f: (e, 0, f)),
            pl.BlockSpec((1, fb, d), lambda e, i, f: (e, f, 0)),
        ],
        out_specs=pl.BlockSpec((1, cb, d), lambda e, i, f: (e, i, 0)),
        out_shape=jax.ShapeDtypeStruct((e, c, d), jnp.float32),
        scratch_shapes=[pltpu.VMEM((cb, d), jnp.float32)],
        compiler_params=pltpu.CompilerParams(
            dimension_semantics=("arbitrary", "arbitrary", "arbitrary"),
        ),
        interpret=interpret,
    )(routed, scores.reshape(e, 1, c), w1b, w2b)


def kernel(x, w_router, w1, w2):
    bz, slen, dim = x.shape
    xf = x.reshape(bz * slen, dim)
    n_tokens = xf.shape[0]
    capacity = (n_tokens * _TOP_K) // _E

    logits = xf @ w_router
    scores = jax.nn.softmax(logits, axis=-1)
    top_scores, selected = jax.lax.top_k(scores.T, capacity)
    token_indices = selected.reshape(-1)

    routed = xf.at[token_indices].get(
        mode="promise_in_bounds").reshape(_E, capacity, dim)
    routed_out = _ffn(routed, top_scores, w1, w2)

    out = jnp.zeros_like(xf)
    out = out.at[token_indices].add(routed_out.reshape(-1, dim))
    return out.reshape(bz, slen, dim)


# speedup vs baseline: 1.2698x; 1.2698x over previous
import functools

import jax
import jax.numpy as jnp
from jax.experimental import pallas as pl
from jax.experimental.pallas import tpu as pltpu

_E = 8
_TOP_K = 2

_FB1 = 256
_CB2 = 2048
_FB2 = 256


def _mm1_body(x_ref, s_ref, w1_ref, h_ref):
    h = jnp.dot(x_ref[0], w1_ref[0].astype(jnp.bfloat16),
                preferred_element_type=jnp.float32)
    h = h * s_ref[0, 0][:, None]
    h = h * jax.nn.sigmoid(h)
    h_ref[0] = h.astype(jnp.bfloat16)


def _mm2_body(h_ref, w2_ref, o_ref, acc_ref):
    f = pl.program_id(2)
    nf = pl.num_programs(2)
    part = jnp.dot(h_ref[0], w2_ref[0].astype(jnp.bfloat16),
                   preferred_element_type=jnp.float32)

    @pl.when(f == 0)
    def _():
        acc_ref[...] = part

    @pl.when(f != 0)
    def _():
        acc_ref[...] += part

    @pl.when(f == nf - 1)
    def _():
        o_ref[0] = acc_ref[...].astype(jnp.bfloat16)


def _ffn(routed, scores, w1, w2, *, interpret=False):
    e, c, d = routed.shape
    dff = w1.shape[2]
    fb1 = min(_FB1, dff)
    fb2 = min(_FB2, dff)

    h = pl.pallas_call(
        _mm1_body,
        grid=(e, dff // fb1),
        in_specs=[
            pl.BlockSpec((1, c, d), lambda e, f: (e, 0, 0)),
            pl.BlockSpec((1, 1, c), lambda e, f: (e, 0, 0)),
            pl.BlockSpec((1, d, fb1), lambda e, f: (e, 0, f)),
        ],
        out_specs=pl.BlockSpec((1, c, fb1), lambda e, f: (e, 0, f)),
        out_shape=jax.ShapeDtypeStruct((e, c, dff), jnp.bfloat16),
        compiler_params=pltpu.CompilerParams(
            dimension_semantics=("arbitrary", "arbitrary"),
        ),
        interpret=interpret,
    )(routed, scores.reshape(e, 1, c), w1)

    cb2 = min(_CB2, c)
    out = pl.pallas_call(
        _mm2_body,
        grid=(e, c // cb2, dff // fb2),
        in_specs=[
            pl.BlockSpec((1, cb2, fb2), lambda e, i, f: (e, i, f)),
            pl.BlockSpec((1, fb2, d), lambda e, i, f: (e, f, 0)),
        ],
        out_specs=pl.BlockSpec((1, cb2, d), lambda e, i, f: (e, i, 0)),
        out_shape=jax.ShapeDtypeStruct((e, c, d), jnp.bfloat16),
        scratch_shapes=[pltpu.VMEM((cb2, d), jnp.float32)],
        compiler_params=pltpu.CompilerParams(
            dimension_semantics=("arbitrary", "arbitrary", "arbitrary"),
        ),
        interpret=interpret,
    )(h, w2)
    return out


def kernel(x, w_router, w1, w2):
    bz, slen, dim = x.shape
    xf = x.reshape(bz * slen, dim)
    n_tokens = xf.shape[0]
    capacity = (n_tokens * _TOP_K) // _E

    logits = xf @ w_router
    scores = jax.nn.softmax(logits, axis=-1)
    top_scores, selected = jax.lax.top_k(scores.T, capacity)
    token_indices = selected.reshape(-1)

    xb = xf.astype(jnp.bfloat16)
    routed = xb.at[token_indices].get(
        mode="promise_in_bounds").reshape(_E, capacity, dim)
    routed_out = _ffn(routed, top_scores, w1, w2)

    out = jnp.zeros_like(xf)
    out = out.at[token_indices].add(
        routed_out.reshape(-1, dim).astype(jnp.float32))
    return out.reshape(bz, slen, dim)
